# baseline (speedup 1.0000x reference)
import functools

import jax
import jax.numpy as jnp
from jax import lax
from jax.experimental import pallas as pl
from jax.experimental.pallas import tpu as pltpu
from jax.experimental.pallas import tpu_sc as plsc

D = 64
L = 16
C = 128
MAX_NORM = 1.0
_MAGIC = 0x5F3759DF


def _rsqrt(x):
    i = lax.bitcast_convert_type(x, jnp.int32)
    i = _MAGIC - lax.shift_right_logical(i, 1)
    y = lax.bitcast_convert_type(i, jnp.float32)
    for _ in range(3):
        y = y * (jnp.float32(1.5) - jnp.float32(0.5) * x * y * y)
    return y


def _fix_chunk(buf):
    def fix_row(r, carry):
        vs = [buf[r, pl.ds(k * L, L)] for k in range(D // L)]
        sq = vs[0] * vs[0]
        for k in range(1, D // L):
            sq = sq + vs[k] * vs[k]
        ss = jnp.sum(sq)
        rs = _rsqrt(jnp.maximum(ss, jnp.float32(0.25)))
        s = jnp.where(ss > jnp.float32(MAX_NORM * MAX_NORM),
                      jnp.float32(MAX_NORM) * rs, jnp.float32(1.0))
        for k in range(D // L):
            buf[r, pl.ds(k * L, L)] = vs[k] * s
        return carry

    lax.fori_loop(0, C, fix_row, 0)


def _screen_chunk(buf):
    def row_max(r, acc):
        for k in range(D // L):
            v = buf[r, pl.ds(k * L, L)]
            acc = jnp.maximum(acc, jnp.abs(v))
        return acc

    acc = lax.fori_loop(0, C, row_max, jnp.zeros((L,), jnp.float32))
    m = jnp.max(acc)

    @pl.when(m * m * jnp.float32(D) > jnp.float32(MAX_NORM * MAX_NORM))
    def _():
        _fix_chunk(buf)


@functools.lru_cache(maxsize=None)
def _make_sc_embed(nw, nch, vocab):
    mesh = plsc.VectorSubcoreMesh(core_axis_name="c", subcore_axis_name="s")

    @functools.partial(
        pl.kernel,
        mesh=mesh,
        compiler_params=pltpu.CompilerParams(
            needs_layout_passes=False, use_tc_tiling_on_sc=False
        ),
        out_type=jax.ShapeDtypeStruct((nw * nch, C, D), jnp.float32),
        scratch_types=[
            pltpu.VMEM((nch, C), jnp.int32),
            pltpu.VMEM((C, D), jnp.float32),
            pltpu.SemaphoreType.DMA,
        ],
    )
    def sc_embed(tok_hbm, w_hbm, out_hbm, idx_all, buf, gsem):
        nc = plsc.get_sparse_core_info().num_cores
        wid = lax.axis_index("s") * nc + lax.axis_index("c")
        pltpu.sync_copy(tok_hbm.at[wid], idx_all)

        def chunk(j, carry):
            pltpu.async_copy(w_hbm.at[idx_all.at[j]], buf, gsem).wait()
            _screen_chunk(buf)
            pltpu.sync_copy(buf, out_hbm.at[wid * nch + j])
            return carry

        lax.fori_loop(0, nch, chunk, 0)

    return sc_embed


def kernel(token_ids, weight):
    b, t = token_ids.shape
    vocab, d = weight.shape
    assert d == D
    n = b * t
    nw = 32
    per_w = n // nw
    nch = per_w // C
    assert nw * nch * C == n
    tok = token_ids.reshape(nw, nch, C).astype(jnp.int32)
    out = _make_sc_embed(nw, nch, vocab)(tok, weight)
    return out.reshape(b, t, D)


# speedup vs baseline: 1.1884x; 1.1884x over previous
import functools

import jax
import jax.numpy as jnp
from jax import lax
from jax.experimental import pallas as pl
from jax.experimental.pallas import tpu as pltpu
from jax.experimental.pallas import tpu_sc as plsc

D = 64
L = 16
C = 128
NBUF = 4
MAX_NORM = 1.0
_MAGIC = 0x5F3759DF


def _rsqrt(x):
    i = lax.bitcast_convert_type(x, jnp.int32)
    i = _MAGIC - lax.shift_right_logical(i, 1)
    y = lax.bitcast_convert_type(i, jnp.float32)
    for _ in range(3):
        y = y * (jnp.float32(1.5) - jnp.float32(0.5) * x * y * y)
    return y


def _fix_chunk(buf):
    def fix_row(r, carry):
        vs = [buf[r, pl.ds(k * L, L)] for k in range(D // L)]
        sq = vs[0] * vs[0]
        for k in range(1, D // L):
            sq = sq + vs[k] * vs[k]
        ss = jnp.sum(sq)
        rs = _rsqrt(jnp.maximum(ss, jnp.float32(0.25)))
        s = jnp.where(ss > jnp.float32(MAX_NORM * MAX_NORM),
                      jnp.float32(MAX_NORM) * rs, jnp.float32(1.0))
        for k in range(D // L):
            buf[r, pl.ds(k * L, L)] = vs[k] * s
        return carry

    lax.fori_loop(0, C, fix_row, 0)


def _screen_chunk(buf):
    def row_max(r, acc):
        for k in range(D // L):
            v = buf[r, pl.ds(k * L, L)]
            acc = jnp.maximum(acc, jnp.abs(v))
        return acc

    acc = lax.fori_loop(0, C, row_max, jnp.zeros((L,), jnp.float32), unroll=4)
    m = jnp.max(acc)

    @pl.when(m * m * jnp.float32(D) > jnp.float32(MAX_NORM * MAX_NORM))
    def _():
        _fix_chunk(buf)


@functools.lru_cache(maxsize=None)
def _make_sc_embed(nw, nch, vocab):
    mesh = plsc.VectorSubcoreMesh(core_axis_name="c", subcore_axis_name="s")

    @functools.partial(
        pl.kernel,
        mesh=mesh,
        compiler_params=pltpu.CompilerParams(
            needs_layout_passes=False, use_tc_tiling_on_sc=False
        ),
        out_type=jax.ShapeDtypeStruct((nw * nch, C, D), jnp.float32),
        scratch_types=(
            [pltpu.VMEM((nch, C), jnp.int32)]
            + [pltpu.VMEM((C, D), jnp.float32) for _ in range(NBUF)]
            + [pltpu.SemaphoreType.DMA for _ in range(2 * NBUF)]
        ),
    )
    def sc_embed(tok_hbm, w_hbm, out_hbm, idx_all, *bufs_sems):
        bufs = bufs_sems[:NBUF]
        gsem = bufs_sems[NBUF:2 * NBUF]
        ssem = bufs_sems[2 * NBUF:]
        nc = plsc.get_sparse_core_info().num_cores
        wid = lax.axis_index("s") * nc + lax.axis_index("c")
        pltpu.sync_copy(tok_hbm.at[wid], idx_all)

        def gd(j, b):
            return pltpu.make_async_copy(w_hbm.at[idx_all.at[j]], bufs[b], gsem[b])

        def sd(j, b):
            return pltpu.make_async_copy(bufs[b], out_hbm.at[wid * nch + j], ssem[b])

        for b in range(NBUF):
            gd(b, b).start()

        def step(j, b, refill):
            gd(j, b).wait()
            _screen_chunk(bufs[b])
            sd(j, b).start()
            if refill:
                sd(j, b).wait()
                gd(j + NBUF, b).start()

        def ring(i, carry):
            for b in range(NBUF):
                step(i * NBUF + b, b, True)
            return carry

        lax.fori_loop(0, nch // NBUF - 1, ring, 0)
        for b in range(NBUF):
            step(nch - NBUF + b, b, False)
        for b in range(NBUF):
            sd(nch - NBUF + b, b).wait()

    return sc_embed


def kernel(token_ids, weight):
    b, t = token_ids.shape
    vocab, d = weight.shape
    assert d == D
    n = b * t
    nw = 32
    per_w = n // nw
    nch = per_w // C
    assert nw * nch * C == n and nch % NBUF == 0
    tok = token_ids.reshape(nw, nch, C).astype(jnp.int32)
    out = _make_sc_embed(nw, nch, vocab)(tok, weight)
    return out.reshape(b, t, D)


# speedup vs baseline: 1.1924x; 1.0033x over previous
import functools

import jax
import jax.numpy as jnp
from jax import lax
from jax.experimental import pallas as pl
from jax.experimental.pallas import tpu as pltpu
from jax.experimental.pallas import tpu_sc as plsc

D = 64
L = 16
NBUF = 4
MAX_NORM = 1.0
_MAGIC = 0x5F3759DF


def _rsqrt(x):
    i = lax.bitcast_convert_type(x, jnp.int32)
    i = _MAGIC - lax.shift_right_logical(i, 1)
    y = lax.bitcast_convert_type(i, jnp.float32)
    for _ in range(3):
        y = y * (jnp.float32(1.5) - jnp.float32(0.5) * x * y * y)
    return y


def _fix_chunk(buf, nrow):
    def fix_row(r, carry):
        vs = [buf[r, pl.ds(k * L, L)] for k in range(D // L)]
        sq = vs[0] * vs[0]
        for k in range(1, D // L):
            sq = sq + vs[k] * vs[k]
        ss = jnp.sum(sq)
        rs = _rsqrt(jnp.maximum(ss, jnp.float32(0.25)))
        s = jnp.where(ss > jnp.float32(MAX_NORM * MAX_NORM),
                      jnp.float32(MAX_NORM) * rs, jnp.float32(1.0))
        for k in range(D // L):
            buf[r, pl.ds(k * L, L)] = vs[k] * s
        return carry

    lax.fori_loop(0, nrow, fix_row, 0)


def _screen_chunk(buf, nrow):
    def row_max(r, acc):
        for k in range(D // L):
            v = buf[r, pl.ds(k * L, L)]
            acc = jnp.maximum(acc, jnp.abs(v))
        return acc

    acc = lax.fori_loop(0, nrow, row_max, jnp.zeros((L,), jnp.float32), unroll=4)
    m = jnp.max(acc)

    @pl.when(m * m * jnp.float32(D) > jnp.float32(MAX_NORM * MAX_NORM))
    def _():
        _fix_chunk(buf, nrow)


@functools.lru_cache(maxsize=None)
def _make_sc_embed(bsz, seq, vocab):
    nw = 32
    rows_per_w = bsz // nw
    cut = min(128, (seq // 2 + 7) // 8 * 8)
    parts = [(0, cut), (cut, seq - cut)]
    mesh = plsc.VectorSubcoreMesh(core_axis_name="c", subcore_axis_name="s")

    @functools.partial(
        pl.kernel,
        mesh=mesh,
        compiler_params=pltpu.CompilerParams(
            needs_layout_passes=False, use_tc_tiling_on_sc=False
        ),
        out_type=jax.ShapeDtypeStruct((bsz, seq, D), jnp.float32),
        scratch_types=(
            [pltpu.VMEM((rows_per_w, seq), jnp.int32)]
            + [pltpu.VMEM((seq, D), jnp.float32) for _ in range(NBUF)]
            + [pltpu.SemaphoreType.DMA for _ in range(2 * NBUF)]
        ),
    )
    def sc_embed(tok_hbm, w_hbm, out_hbm, idx_all, *bufs_sems):
        bufs = bufs_sems[:NBUF]
        gsem = bufs_sems[NBUF:2 * NBUF]
        ssem = bufs_sems[2 * NBUF:]
        nc = plsc.get_sparse_core_info().num_cores
        wid = lax.axis_index("s") * nc + lax.axis_index("c")
        row0 = wid * rows_per_w
        pltpu.sync_copy(tok_hbm.at[pl.ds(row0, rows_per_w)], idx_all)

        def gd(j, b, h):
            off, ln = parts[h]
            return pltpu.make_async_copy(
                w_hbm.at[idx_all.at[j, pl.ds(off, ln)]],
                bufs[b].at[pl.ds(off, ln)],
                gsem[b],
            )

        def sd(j, b):
            return pltpu.make_async_copy(bufs[b], out_hbm.at[row0 + j], ssem[b])

        for b in range(NBUF):
            gd(b, b, 0).start()
            gd(b, b, 1).start()

        def step(j, b, refill):
            gd(j, b, 0).wait()
            gd(j, b, 1).wait()
            _screen_chunk(bufs[b], seq)
            sd(j, b).start()
            if refill:
                sd(j, b).wait()
                gd(j + NBUF, b, 0).start()
                gd(j + NBUF, b, 1).start()

        def ring(i, carry):
            for b in range(NBUF):
                step(i * NBUF + b, b, True)
            return carry

        lax.fori_loop(0, rows_per_w // NBUF - 1, ring, 0)
        for b in range(NBUF):
            step(rows_per_w - NBUF + b, b, False)
        for b in range(NBUF):
            sd(rows_per_w - NBUF + b, b).wait()

    return sc_embed


def kernel(token_ids, weight):
    bsz, seq = token_ids.shape
    vocab, d = weight.shape
    assert d == D and bsz % (32 * NBUF) == 0 and seq % 8 == 0 and seq <= 248
    return _make_sc_embed(bsz, seq, vocab)(token_ids.astype(jnp.int32), weight)


# speedup vs baseline: 1.5802x; 1.3253x over previous
import functools

import jax
import jax.numpy as jnp
from jax import lax
from jax.experimental import pallas as pl
from jax.experimental.pallas import tpu as pltpu
from jax.experimental.pallas import tpu_sc as plsc

D = 64
L = 16
NBUF = 4
MAX_NORM = 1.0
_MAGIC = 0x5F3759DF


def _rsqrt(x):
    i = lax.bitcast_convert_type(x, jnp.int32)
    i = _MAGIC - lax.shift_right_logical(i, 1)
    y = lax.bitcast_convert_type(i, jnp.float32)
    for _ in range(3):
        y = y * (jnp.float32(1.5) - jnp.float32(0.5) * x * y * y)
    return y


def _fix_chunk(buf, nrow):
    def fix_row(r, carry):
        vs = [buf[r, pl.ds(k * L, L)] for k in range(D // L)]
        sq = vs[0] * vs[0]
        for k in range(1, D // L):
            sq = sq + vs[k] * vs[k]
        ss = jnp.sum(sq)
        rs = _rsqrt(jnp.maximum(ss, jnp.float32(0.25)))
        s = jnp.where(ss > jnp.float32(MAX_NORM * MAX_NORM),
                      jnp.float32(MAX_NORM) * rs, jnp.float32(1.0))
        for k in range(D // L):
            buf[r, pl.ds(k * L, L)] = vs[k] * s
        return carry

    lax.fori_loop(0, nrow, fix_row, 0)


def _screen_chunk(buf, nrow):
    def row_max(r, acc):
        for k in range(D // L):
            v = buf[r, pl.ds(k * L, L)]
            acc = jnp.maximum(acc, jnp.abs(v))
        return acc

    acc = lax.fori_loop(0, nrow, row_max, jnp.zeros((L,), jnp.float32), unroll=4)
    m = jnp.max(acc)

    @pl.when(m * m * jnp.float32(D) > jnp.float32(MAX_NORM * MAX_NORM))
    def _():
        _fix_chunk(buf, nrow)


@functools.lru_cache(maxsize=None)
def _make_sc_embed(bsz, seq, vocab):
    nw = 32
    rows_per_w = bsz // nw
    cut = min(128, (seq // 2 + 7) // 8 * 8)
    parts = [(0, cut), (cut, seq - cut)]
    mesh = plsc.VectorSubcoreMesh(core_axis_name="c", subcore_axis_name="s")

    @functools.partial(
        pl.kernel,
        mesh=mesh,
        compiler_params=pltpu.CompilerParams(
            needs_layout_passes=False, use_tc_tiling_on_sc=False
        ),
        out_type=jax.ShapeDtypeStruct((bsz * seq, 2 * D), jnp.float32),
        scratch_types=(
            [pltpu.VMEM((rows_per_w, seq), jnp.int32)]
            + [pltpu.VMEM((seq, D), jnp.float32) for _ in range(NBUF)]
            + [pltpu.SemaphoreType.DMA for _ in range(2 * NBUF)]
        ),
    )
    def sc_embed(tok_hbm, w_hbm, out_hbm, idx_all, *bufs_sems):
        bufs = bufs_sems[:NBUF]
        gsem = bufs_sems[NBUF:2 * NBUF]
        ssem = bufs_sems[2 * NBUF:]
        nc = plsc.get_sparse_core_info().num_cores
        wid = lax.axis_index("s") * nc + lax.axis_index("c")
        row0 = wid * rows_per_w
        pltpu.sync_copy(tok_hbm.at[pl.ds(row0, rows_per_w)], idx_all)

        def gd(j, b, h):
            off, ln = parts[h]
            return pltpu.make_async_copy(
                w_hbm.at[idx_all.at[j, pl.ds(off, ln)]],
                bufs[b].at[pl.ds(off, ln)],
                gsem[b],
            )

        def sd(j, b):
            return pltpu.make_async_copy(
                bufs[b],
                out_hbm.at[pl.ds((row0 + j) * seq, seq), pl.ds(0, D)],
                ssem[b],
            )

        for b in range(NBUF):
            gd(b, b, 0).start()
            gd(b, b, 1).start()

        def step(j, b, refill):
            gd(j, b, 0).wait()
            gd(j, b, 1).wait()
            _screen_chunk(bufs[b], seq)
            sd(j, b).start()
            if refill:
                sd(j, b).wait()
                gd(j + NBUF, b, 0).start()
                gd(j + NBUF, b, 1).start()

        def ring(i, carry):
            for b in range(NBUF):
                step(i * NBUF + b, b, True)
            return carry

        lax.fori_loop(0, rows_per_w // NBUF - 1, ring, 0)
        for b in range(NBUF):
            step(rows_per_w - NBUF + b, b, False)
        for b in range(NBUF):
            sd(rows_per_w - NBUF + b, b).wait()

    return sc_embed


def kernel(token_ids, weight):
    bsz, seq = token_ids.shape
    vocab, d = weight.shape
    assert d == D and bsz % (32 * NBUF) == 0 and seq % 8 == 0 and seq <= 248
    w_dense = lax.optimization_barrier(weight.reshape(-1)).reshape(vocab, D)
    out = _make_sc_embed(bsz, seq, vocab)(token_ids.astype(jnp.int32), w_dense)
    return out[:, :D].reshape(bsz, seq, D)


# speedup vs baseline: 1.8568x; 1.1750x over previous
import functools

import jax
import jax.numpy as jnp
from jax import lax
from jax.experimental import pallas as pl
from jax.experimental.pallas import tpu as pltpu
from jax.experimental.pallas import tpu_sc as plsc

D = 64
L = 16
NBUF = 4
MAX_NORM = 1.0
_MAGIC = 0x5F3759DF


def _rsqrt(x):
    i = lax.bitcast_convert_type(x, jnp.int32)
    i = _MAGIC - lax.shift_right_logical(i, 1)
    y = lax.bitcast_convert_type(i, jnp.float32)
    for _ in range(3):
        y = y * (jnp.float32(1.5) - jnp.float32(0.5) * x * y * y)
    return y


def _fix_chunk(buf, nrow):
    def fix_row(r, carry):
        vs = [buf[r, pl.ds(k * L, L)] for k in range(D // L)]
        sq = vs[0] * vs[0]
        for k in range(1, D // L):
            sq = sq + vs[k] * vs[k]
        ss = jnp.sum(sq)
        rs = _rsqrt(jnp.maximum(ss, jnp.float32(0.25)))
        s = jnp.where(ss > jnp.float32(MAX_NORM * MAX_NORM),
                      jnp.float32(MAX_NORM) * rs, jnp.float32(1.0))
        for k in range(D // L):
            buf[r, pl.ds(k * L, L)] = vs[k] * s
        return carry

    lax.fori_loop(0, nrow, fix_row, 0)


def _screen_chunk(buf, nrow):
    def row_max(r, acc):
        for k in range(D // L):
            v = buf[r, pl.ds(k * L, L)]
            acc = jnp.maximum(acc, jnp.abs(v))
        return acc

    acc = lax.fori_loop(0, nrow, row_max, jnp.zeros((L,), jnp.float32), unroll=4)
    m = jnp.max(acc)

    @pl.when(m * m * jnp.float32(D) > jnp.float32(MAX_NORM * MAX_NORM))
    def _():
        _fix_chunk(buf, nrow)


@functools.lru_cache(maxsize=None)
def _make_tc_detile(vocab):
    br = 1024
    nblk = (vocab + 2 * br - 1) // (2 * br)
    ub = (vocab + br - 1) // br - 1

    def body(xlo_ref, xhi_ref, o_ref):
        o_ref[:, 0:D] = xlo_ref[...].T
        o_ref[:, D:2 * D] = xhi_ref[...].T

    return pl.pallas_call(
        body,
        grid=(nblk,),
        in_specs=[
            pl.BlockSpec((D, br), lambda i: (0, jnp.minimum(2 * i, ub))),
            pl.BlockSpec((D, br), lambda i: (0, jnp.minimum(2 * i + 1, ub))),
        ],
        out_specs=pl.BlockSpec((br, 2 * D), lambda i: (i, 0)),
        out_shape=jax.ShapeDtypeStruct((nblk * br, 2 * D), jnp.float32),
    )


@functools.lru_cache(maxsize=None)
def _make_sc_embed(bsz, seq, vocab):
    nw = 32
    rows_per_w = bsz // nw
    cut = min(128, (seq // 2 + 7) // 8 * 8)
    parts = [(0, cut), (cut, seq - cut)]
    mesh = plsc.VectorSubcoreMesh(core_axis_name="c", subcore_axis_name="s")

    @functools.partial(
        pl.kernel,
        mesh=mesh,
        compiler_params=pltpu.CompilerParams(
            needs_layout_passes=False, use_tc_tiling_on_sc=False
        ),
        out_type=jax.ShapeDtypeStruct((bsz * seq, 2 * D), jnp.float32),
        scratch_types=(
            [pltpu.VMEM((rows_per_w, seq), jnp.int32)]
            + [pltpu.VMEM((seq, D), jnp.float32) for _ in range(NBUF)]
            + [pltpu.SemaphoreType.DMA for _ in range(2 * NBUF)]
        ),
    )
    def sc_embed(tok_hbm, w_hbm, out_hbm, idx_all, *bufs_sems):
        bufs = bufs_sems[:NBUF]
        gsem = bufs_sems[NBUF:2 * NBUF]
        ssem = bufs_sems[2 * NBUF:]
        nc = plsc.get_sparse_core_info().num_cores
        wid = lax.axis_index("s") * nc + lax.axis_index("c")
        row0 = wid * rows_per_w
        pltpu.sync_copy(tok_hbm.at[pl.ds(row0, rows_per_w)], idx_all)

        def gd(j, b, h):
            off, ln = parts[h]
            return pltpu.make_async_copy(
                w_hbm.at[idx_all.at[j, pl.ds(off, ln)]],
                bufs[b].at[pl.ds(off, ln)],
                gsem[b],
            )

        def sd(j, b):
            return pltpu.make_async_copy(
                bufs[b],
                out_hbm.at[pl.ds((row0 + j) * seq, seq), pl.ds(0, D)],
                ssem[b],
            )

        for b in range(NBUF):
            gd(b, b, 0).start()
            gd(b, b, 1).start()

        def step(j, b, refill):
            gd(j, b, 0).wait()
            gd(j, b, 1).wait()
            _screen_chunk(bufs[b], seq)
            sd(j, b).start()
            if refill:
                sd(j, b).wait()
                gd(j + NBUF, b, 0).start()
                gd(j + NBUF, b, 1).start()

        def ring(i, carry):
            for b in range(NBUF):
                step(i * NBUF + b, b, True)
            return carry

        lax.fori_loop(0, rows_per_w // NBUF - 1, ring, 0)
        for b in range(NBUF):
            step(rows_per_w - NBUF + b, b, False)
        for b in range(NBUF):
            sd(rows_per_w - NBUF + b, b).wait()

    return sc_embed


def kernel(token_ids, weight):
    bsz, seq = token_ids.shape
    vocab, d = weight.shape
    assert d == D and bsz % (32 * NBUF) == 0 and seq % 8 == 0 and seq <= 248
    wt = weight.T
    w_pairs = _make_tc_detile(vocab)(wt, wt)
    vvocab = 2 * w_pairs.shape[0]
    w_dense = w_pairs.reshape(vvocab, D)
    tok = token_ids.astype(jnp.int32)
    tok = 2 * ((tok >> 11) * 1024 + (tok & 1023)) + ((tok & 2047) >> 10)
    out = _make_sc_embed(bsz, seq, vvocab)(tok, w_dense)
    return out[:, :D].reshape(bsz, seq, D)
